# Initial kernel scaffold; baseline (speedup 1.0000x reference)
import jax
import jax.numpy as jnp
from jax.experimental import pallas as pl


def kernel(x, edge_index, edge_emb, l_weight, root, message_bias):
    raise NotImplementedError("write your pallas kernel here")


# speedup vs baseline: 3.0204x; 3.0204x over previous
import functools

import jax
import jax.numpy as jnp
from jax import lax
from jax.experimental import pallas as pl
from jax.experimental.pallas import tpu as pltpu
from jax.experimental.pallas import tpu_sc as plsc

N = 10000
E = 320000
D = 128

NC = 2
NS = 16
NW = NC * NS
EPW = E // NW
CH = 80
NCH = EPW // CH
NPAD = 10240
NPS = NPAD // NS

_mesh = plsc.VectorSubcoreMesh(core_axis_name="c", subcore_axis_name="s")


@functools.partial(
    pl.kernel,
    out_type=jax.ShapeDtypeStruct((E, D), jnp.float32),
    mesh=_mesh,
    scratch_types=[
        pltpu.VMEM((CH,), jnp.int32),
        pltpu.VMEM((CH, D), jnp.float32),
        pltpu.SemaphoreType.DMA,
    ],
)
def _gather_xj(x_hbm, src_hbm, xj_hbm, idx_v, rows_v, sem):
    wid = lax.axis_index("s") * NC + lax.axis_index("c")
    base = wid * EPW

    def body(i, carry):
        off = base + i * CH
        pltpu.sync_copy(src_hbm.at[pl.ds(off, CH)], idx_v)
        pltpu.async_copy(x_hbm.at[idx_v], rows_v, sem).wait()
        pltpu.sync_copy(rows_v, xj_hbm.at[pl.ds(off, CH)])
        return carry

    lax.fori_loop(0, NCH, body, 0)


def _msg_body(emb_ref, xj_ref, w_ref, m_ref):
    rw = jnp.dot(emb_ref[...], w_ref[...], preferred_element_type=jnp.float32)
    m_ref[...] = rw * xj_ref[...]


_BE = 2000


def _msg(edge_emb, xj, l_weight):
    return pl.pallas_call(
        _msg_body,
        grid=(E // _BE,),
        in_specs=[
            pl.BlockSpec((_BE, D), lambda i: (i, 0)),
            pl.BlockSpec((_BE, D), lambda i: (i, 0)),
            pl.BlockSpec((D, D), lambda i: (0, 0)),
        ],
        out_specs=pl.BlockSpec((_BE, D), lambda i: (i, 0)),
        out_shape=jax.ShapeDtypeStruct((E, D), jnp.float32),
        compiler_params=pltpu.CompilerParams(
            dimension_semantics=("arbitrary",)),
    )(edge_emb, xj, l_weight)


@functools.partial(
    pl.kernel,
    out_type=jax.ShapeDtypeStruct((NC * NPAD, D), jnp.float32),
    mesh=_mesh,
    scratch_types=[
        pltpu.VMEM((CH,), jnp.int32),
        pltpu.VMEM((CH, D), jnp.float32),
        pltpu.VMEM_SHARED((NPAD, D), jnp.float32),
        pltpu.SemaphoreType.DMA,
    ],
)
def _scatter_sums(m_hbm, dst_hbm, za_hbm, pa_hbm, idx_v, m_v, acc_a, sem):
    c = lax.axis_index("c")
    s = lax.axis_index("s")
    base = (s * NC + c) * EPW

    pltpu.sync_copy(za_hbm.at[pl.ds(s * NPS, NPS)], acc_a.at[pl.ds(s * NPS, NPS)])
    plsc.subcore_barrier()

    def body(i, carry):
        off = base + i * CH
        pltpu.sync_copy(dst_hbm.at[pl.ds(off, CH)], idx_v)
        pltpu.sync_copy(m_hbm.at[pl.ds(off, CH)], m_v)
        pltpu.sync_copy(m_v, acc_a.at[idx_v], add=True)
        return carry

    lax.fori_loop(0, NCH, body, 0)
    plsc.subcore_barrier()

    out_row = c * NPAD + s * NPS
    pltpu.sync_copy(acc_a.at[pl.ds(s * NPS, NPS)], pa_hbm.at[pl.ds(out_row, NPS)])


@functools.partial(
    pl.kernel,
    out_type=jax.ShapeDtypeStruct((NC * NPAD, D), jnp.float32),
    mesh=_mesh,
    scratch_types=[
        pltpu.VMEM((CH,), jnp.int32),
        pltpu.VMEM((CH, D), jnp.float32),
        pltpu.VMEM_SHARED((NPAD, D), jnp.float32),
        pltpu.SemaphoreType.DMA,
    ],
)
def _scatter_counts(dst_hbm, za_hbm, ones_hbm, pc_hbm, idx_v, ones_v, acc_c, sem):
    c = lax.axis_index("c")
    s = lax.axis_index("s")
    base = (s * NC + c) * EPW

    pltpu.sync_copy(za_hbm.at[pl.ds(s * NPS, NPS)], acc_c.at[pl.ds(s * NPS, NPS)])
    pltpu.sync_copy(ones_hbm, ones_v)
    plsc.subcore_barrier()

    def body(i, carry):
        off = base + i * CH
        pltpu.sync_copy(dst_hbm.at[pl.ds(off, CH)], idx_v)
        pltpu.sync_copy(ones_v, acc_c.at[idx_v], add=True)
        return carry

    lax.fori_loop(0, NCH, body, 0)
    plsc.subcore_barrier()

    out_row = c * NPAD + s * NPS
    pltpu.sync_copy(acc_c.at[pl.ds(s * NPS, NPS)], pc_hbm.at[pl.ds(out_row, NPS)])


_BN = 1024


def _combine_body(pa_ref, pc_ref, x_ref, root_ref, bias_ref, out_ref):
    ssum = pa_ref[0] + pa_ref[1]
    cnt = pc_ref[0][:, 0:1] + pc_ref[1][:, 0:1]
    xr = jnp.dot(x_ref[...], root_ref[...], preferred_element_type=jnp.float32)
    out_ref[...] = ssum / jnp.maximum(cnt, 1.0) + xr + bias_ref[...]


def _combine(pa, pc, x, root, bias2d):
    return pl.pallas_call(
        _combine_body,
        grid=(NPAD // _BN,),
        in_specs=[
            pl.BlockSpec((NC, _BN, D), lambda i: (0, i, 0)),
            pl.BlockSpec((NC, _BN, D), lambda i: (0, i, 0)),
            pl.BlockSpec((_BN, D), lambda i: (i, 0)),
            pl.BlockSpec((D, D), lambda i: (0, 0)),
            pl.BlockSpec((1, D), lambda i: (0, 0)),
        ],
        out_specs=pl.BlockSpec((_BN, D), lambda i: (i, 0)),
        out_shape=jax.ShapeDtypeStruct((NPAD, D), jnp.float32),
        compiler_params=pltpu.CompilerParams(
            dimension_semantics=("arbitrary",)),
    )(pa, pc, x, root, bias2d)


def kernel(x, edge_index, edge_emb, l_weight, root, message_bias):
    src = edge_index[0]
    dst = edge_index[1]
    za = jnp.zeros((NPAD, D), jnp.float32)
    ones = jnp.ones((CH, D), jnp.float32)

    xj = _gather_xj(x, src)
    m = _msg(edge_emb, xj, l_weight)
    pa = _scatter_sums(m, dst, za)
    pc = _scatter_counts(dst, za, ones)
    xp = jnp.concatenate([x, jnp.zeros((NPAD - N, D), jnp.float32)], axis=0)
    out = _combine(pa.reshape(NC, NPAD, D), pc.reshape(NC, NPAD, D),
                   xp, root, message_bias.reshape(1, D))
    return out[:N]


# speedup vs baseline: 4.8173x; 1.5949x over previous
import functools

import jax
import jax.numpy as jnp
from jax import lax
from jax.experimental import pallas as pl
from jax.experimental.pallas import tpu as pltpu
from jax.experimental.pallas import tpu_sc as plsc

N = 10000
E = 320000
D = 128

NC = 2
NS = 16
NW = NC * NS
EPW = E // NW
CH = 80
NCH = EPW // CH
NB = 5
NJ = NCH // NB
NPAD = 10240
NPS = NPAD // NS
CHS = 40
NCHS = EPW // CHS
NJS = NCHS // NB

_mesh = plsc.VectorSubcoreMesh(core_axis_name="c", subcore_axis_name="s")


@functools.partial(
    pl.kernel,
    out_type=jax.ShapeDtypeStruct((E, D), jnp.float32),
    mesh=_mesh,
    scratch_types=[
        pltpu.VMEM((EPW,), jnp.int32),
        pltpu.VMEM((NB, CH, D), jnp.float32),
        pltpu.SemaphoreType.DMA((NB,)),
        pltpu.SemaphoreType.DMA((NB,)),
    ],
)
def _gather_xj(x_hbm, src_hbm, xj_hbm, idx_all, rows_v, gsem, ssem):
    wid = lax.axis_index("s") * NC + lax.axis_index("c")
    base = wid * EPW
    pltpu.sync_copy(src_hbm.at[pl.ds(base, EPW)], idx_all)

    def gath(i, k):
        return pltpu.make_async_copy(
            x_hbm.at[idx_all.at[pl.ds(i * CH, CH)]], rows_v.at[k], gsem.at[k])

    def store(i, k):
        return pltpu.make_async_copy(
            rows_v.at[k], xj_hbm.at[pl.ds(base + i * CH, CH)], ssem.at[k])

    def body(j, carry):
        for k in range(NB):
            i = j * NB + k

            @pl.when(j >= 1)
            def _():
                store(i - NB, k).wait()

            gath(i, k).start()
        for k in range(NB):
            i = j * NB + k
            gath(i, k).wait()
            store(i, k).start()
        return carry

    lax.fori_loop(0, NJ, body, 0)
    for k in range(NB):
        store((NJ - 1) * NB + k, k).wait()


def _msg_body(emb_ref, xj_ref, w_ref, m_ref):
    rw = jnp.dot(emb_ref[...], w_ref[...], preferred_element_type=jnp.float32)
    m_ref[...] = rw * xj_ref[...]


_BE = 2000


def _msg(edge_emb, xj, l_weight):
    return pl.pallas_call(
        _msg_body,
        grid=(E // _BE,),
        in_specs=[
            pl.BlockSpec((_BE, D), lambda i: (i, 0)),
            pl.BlockSpec((_BE, D), lambda i: (i, 0)),
            pl.BlockSpec((D, D), lambda i: (0, 0)),
        ],
        out_specs=pl.BlockSpec((_BE, D), lambda i: (i, 0)),
        out_shape=jax.ShapeDtypeStruct((E, D), jnp.float32),
        compiler_params=pltpu.CompilerParams(
            dimension_semantics=("arbitrary",)),
    )(edge_emb, xj, l_weight)


@functools.partial(
    pl.kernel,
    out_type=jax.ShapeDtypeStruct((NC * NPAD, D), jnp.float32),
    mesh=_mesh,
    scratch_types=[
        pltpu.VMEM((NB, CHS), jnp.int32),
        pltpu.VMEM((NB, CHS, D), jnp.float32),
        pltpu.VMEM_SHARED((NPAD, D), jnp.float32),
        pltpu.SemaphoreType.DMA((NB,)),
        pltpu.SemaphoreType.DMA((NB,)),
        pltpu.SemaphoreType.DMA((NB,)),
    ],
)
def _scatter_sums(m_hbm, dst3_hbm, za_hbm, pa_hbm,
                  idxb, m_v, acc_a, isem, msem, scsem):
    c = lax.axis_index("c")
    s = lax.axis_index("s")
    wid = s * NC + c
    base = wid * EPW

    pltpu.sync_copy(za_hbm.at[pl.ds(s * NPS, NPS)], acc_a.at[pl.ds(s * NPS, NPS)])
    plsc.subcore_barrier()

    def iload(i, k):
        return pltpu.make_async_copy(
            dst3_hbm.at[wid, i], idxb.at[k], isem.at[k])

    def mload(i, k):
        return pltpu.make_async_copy(
            m_hbm.at[pl.ds(base + i * CHS, CHS)], m_v.at[k], msem.at[k])

    def scat(i, k):
        return pltpu.make_async_copy(
            m_v.at[k], acc_a.at[idxb.at[k]], scsem.at[k])

    def body(j, carry):
        for k in range(NB):
            i = j * NB + k

            @pl.when(j >= 1)
            def _():
                scat(i - NB, k).wait()

            iload(i, k).start()
            mload(i, k).start()
        for k in range(NB):
            i = j * NB + k
            iload(i, k).wait()
            mload(i, k).wait()
            pltpu.async_copy(m_v.at[k], acc_a.at[idxb.at[k]],
                             scsem.at[k], add=True)
        return carry

    lax.fori_loop(0, NJS, body, 0)
    for k in range(NB):
        scat((NJS - 1) * NB + k, k).wait()
    plsc.subcore_barrier()

    out_row = c * NPAD + s * NPS
    pltpu.sync_copy(acc_a.at[pl.ds(s * NPS, NPS)], pa_hbm.at[pl.ds(out_row, NPS)])


@functools.partial(
    pl.kernel,
    out_type=jax.ShapeDtypeStruct((NC * NPAD, D), jnp.float32),
    mesh=_mesh,
    scratch_types=[
        pltpu.VMEM((NB, CHS), jnp.int32),
        pltpu.VMEM((CHS, D), jnp.float32),
        pltpu.VMEM_SHARED((NPAD, D), jnp.float32),
        pltpu.SemaphoreType.DMA((NB,)),
        pltpu.SemaphoreType.DMA((NB,)),
    ],
)
def _scatter_counts(dst3_hbm, za_hbm, ones_hbm, pc_hbm,
                    idxb, ones_v, acc_c, isem, csem):
    c = lax.axis_index("c")
    s = lax.axis_index("s")
    wid = s * NC + c

    pltpu.sync_copy(za_hbm.at[pl.ds(s * NPS, NPS)], acc_c.at[pl.ds(s * NPS, NPS)])
    pltpu.sync_copy(ones_hbm, ones_v)
    plsc.subcore_barrier()

    def iload(i, k):
        return pltpu.make_async_copy(
            dst3_hbm.at[wid, i], idxb.at[k], isem.at[k])

    def cscat(i, k):
        return pltpu.make_async_copy(
            ones_v, acc_c.at[idxb.at[k]], csem.at[k])

    def body(j, carry):
        for k in range(NB):
            i = j * NB + k

            @pl.when(j >= 1)
            def _():
                cscat(i - NB, k).wait()

            iload(i, k).start()
        for k in range(NB):
            i = j * NB + k
            iload(i, k).wait()
            pltpu.async_copy(ones_v, acc_c.at[idxb.at[k]],
                             csem.at[k], add=True)
        return carry

    lax.fori_loop(0, NJS, body, 0)
    for k in range(NB):
        cscat((NJS - 1) * NB + k, k).wait()
    plsc.subcore_barrier()

    out_row = c * NPAD + s * NPS
    pltpu.sync_copy(acc_c.at[pl.ds(s * NPS, NPS)], pc_hbm.at[pl.ds(out_row, NPS)])


_BN = 1024


def _combine_body(pa_ref, pc_ref, x_ref, root_ref, bias_ref, out_ref):
    ssum = pa_ref[0] + pa_ref[1]
    cnt = pc_ref[0][:, 0:1] + pc_ref[1][:, 0:1]
    xr = jnp.dot(x_ref[...], root_ref[...], preferred_element_type=jnp.float32)
    out_ref[...] = ssum / jnp.maximum(cnt, 1.0) + xr + bias_ref[...]


def _combine(pa, pc, x, root, bias2d):
    return pl.pallas_call(
        _combine_body,
        grid=(NPAD // _BN,),
        in_specs=[
            pl.BlockSpec((NC, _BN, D), lambda i: (0, i, 0)),
            pl.BlockSpec((NC, _BN, D), lambda i: (0, i, 0)),
            pl.BlockSpec((_BN, D), lambda i: (i, 0)),
            pl.BlockSpec((D, D), lambda i: (0, 0)),
            pl.BlockSpec((1, D), lambda i: (0, 0)),
        ],
        out_specs=pl.BlockSpec((_BN, D), lambda i: (i, 0)),
        out_shape=jax.ShapeDtypeStruct((NPAD, D), jnp.float32),
        compiler_params=pltpu.CompilerParams(
            dimension_semantics=("arbitrary",)),
    )(pa, pc, x, root, bias2d)


def kernel(x, edge_index, edge_emb, l_weight, root, message_bias):
    src = edge_index[0]
    dst3 = edge_index[1].reshape(NW, NCHS, CHS)
    za = jnp.zeros((NPAD, D), jnp.float32)
    ones = jnp.ones((CHS, D), jnp.float32)

    xj = _gather_xj(x, src)
    m = _msg(edge_emb, xj, l_weight)
    pa = _scatter_sums(m, dst3, za)
    pc = _scatter_counts(dst3, za, ones)
    xp = jnp.concatenate([x, jnp.zeros((NPAD - N, D), jnp.float32)], axis=0)
    out = _combine(pa.reshape(NC, NPAD, D), pc.reshape(NC, NPAD, D),
                   xp, root, message_bias.reshape(1, D))
    return out[:N]
